# Initial kernel scaffold; baseline (speedup 1.0000x reference)
import jax
import jax.numpy as jnp
from jax.experimental import pallas as pl


def kernel(x0, x1, edge_index, W1a, b1a, W1b, b1b, W2, b2):
    raise NotImplementedError("write your pallas kernel here")


# speedup vs baseline: 22.2332x; 22.2332x over previous
import functools

import jax
import jax.numpy as jnp
from jax import lax
from jax.experimental import pallas as pl
from jax.experimental.pallas import tpu as pltpu
from jax.experimental.pallas import tpu_sc as plsc

NC = 2
NS = 16
NW = NC * NS
B = 128


def _scatter_sc(width, nch, n_pad):
    rpt = n_pad // NS
    mesh = plsc.VectorSubcoreMesh(core_axis_name="c", subcore_axis_name="s")

    @functools.partial(
        pl.kernel,
        out_type=jax.ShapeDtypeStruct((NC, n_pad, width), jnp.float32),
        mesh=mesh,
        compiler_params=pltpu.CompilerParams(use_tc_tiling_on_sc=(width % 128 == 0)),
        scratch_types=[
            pltpu.VMEM((nch, B), jnp.int32),
            pltpu.VMEM((nch, B), jnp.int32),
            pltpu.VMEM((B, width), jnp.float32),
            pltpu.VMEM_SHARED((n_pad, width), jnp.float32),
            pltpu.SemaphoreType.DMA,
        ],
    )
    def sc_kernel(src_hbm, dst_hbm, vals_hbm, out_hbm, src_v, dst_v, rows_v, acc_sh, sem):
        c = lax.axis_index("c")
        s = lax.axis_index("s")
        wid = c * NS + s

        zeros16 = jnp.zeros((16,), jnp.float32)

        @pl.loop(0, B)
        def _(i):
            for k in range(width // 16):
                rows_v[i, pl.ds(k * 16, 16)] = zeros16

        @pl.loop(0, rpt // B)
        def _(j):
            pltpu.sync_copy(rows_v, acc_sh.at[pl.ds(s * rpt + j * B, B)])

        plsc.subcore_barrier()

        pltpu.sync_copy(src_hbm.at[wid], src_v)
        pltpu.sync_copy(dst_hbm.at[wid], dst_v)

        @pl.loop(0, nch)
        def _(j):
            pltpu.async_copy(vals_hbm.at[src_v.at[j]], rows_v, sem).wait()
            pltpu.sync_copy(rows_v, acc_sh.at[dst_v.at[j]], add=True)

        plsc.subcore_barrier()

        @pl.loop(0, rpt // B)
        def _(j):
            r = s * rpt + j * B
            pltpu.sync_copy(acc_sh.at[pl.ds(r, B)], out_hbm.at[c].at[pl.ds(r, B)])

    return sc_kernel


def _deg_sc(nch, n_pad):
    width = 16
    rpt = n_pad // NS
    mesh = plsc.VectorSubcoreMesh(core_axis_name="c", subcore_axis_name="s")

    @functools.partial(
        pl.kernel,
        out_type=jax.ShapeDtypeStruct((NC, n_pad, width), jnp.float32),
        mesh=mesh,
        scratch_types=[
            pltpu.VMEM((nch, B), jnp.int32),
            pltpu.VMEM((B, width), jnp.float32),
            pltpu.VMEM_SHARED((n_pad, width), jnp.float32),
        ],
    )
    def deg_kernel(dst_hbm, out_hbm, dst_v, ones_v, acc_sh):
        c = lax.axis_index("c")
        s = lax.axis_index("s")
        wid = c * NS + s

        zeros16 = jnp.zeros((16,), jnp.float32)

        @pl.loop(0, B)
        def _(i):
            ones_v[i, pl.ds(0, 16)] = zeros16

        @pl.loop(0, rpt // B)
        def _(j):
            pltpu.sync_copy(ones_v, acc_sh.at[pl.ds(s * rpt + j * B, B)])

        ones16 = jnp.ones((16,), jnp.float32)

        @pl.loop(0, B)
        def _(i):
            ones_v[i, pl.ds(0, 16)] = ones16

        plsc.subcore_barrier()

        pltpu.sync_copy(dst_hbm.at[wid], dst_v)

        @pl.loop(0, nch)
        def _(j):
            pltpu.sync_copy(ones_v, acc_sh.at[dst_v.at[j]], add=True)

        plsc.subcore_barrier()

        @pl.loop(0, rpt // B)
        def _(j):
            r = s * rpt + j * B
            pltpu.sync_copy(acc_sh.at[pl.ds(r, B)], out_hbm.at[c].at[pl.ds(r, B)])

    return deg_kernel


def _tc1_body(x0_ref, x1_ref, w1a_ref, w1b_ref, degp_ref, zs_ref, dinv_ref):
    h = w1a_ref.shape[1]
    deg = degp_ref[0, :, 0:1] + degp_ref[1, :, 0:1] + 1.0
    dinv = lax.rsqrt(deg)
    z0 = jnp.dot(x0_ref[...], w1a_ref[...], preferred_element_type=jnp.float32)
    z1 = jnp.dot(x1_ref[...], w1b_ref[...], preferred_element_type=jnp.float32)
    zs_ref[:, :h] = z0 * dinv
    zs_ref[:, h:] = z1 * dinv
    dinv_ref[...] = dinv


def _tc2_body(p_ref, zs_ref, dinv_ref, b1_ref, w2_ref, gs_ref):
    dinv = dinv_ref[...]
    pre = (p_ref[0] + p_ref[1] + zs_ref[...]) * dinv + b1_ref[...]
    hid = jnp.maximum(pre, 0.0)
    g = jnp.dot(hid, w2_ref[...], preferred_element_type=jnp.float32)
    gs_ref[...] = g * dinv


def _tc3_body(q_ref, gs_ref, dinv_ref, b2_ref, out_ref):
    o = (q_ref[0] + q_ref[1] + gs_ref[...]) * dinv_ref[...] + b2_ref[...]
    m = jnp.max(o, axis=1, keepdims=True)
    e = jnp.exp(o - m)
    lse = jnp.log(jnp.sum(e, axis=1, keepdims=True)) + m
    out_ref[...] = o - lse


def kernel(x0, x1, edge_index, W1a, b1a, W1b, b1b, W2, b2):
    n, d = x0.shape
    e = edge_index.shape[1]
    h = W1a.shape[1]
    c_out = W2.shape[1]

    n_pad = ((n + 1 + NS * B - 1) // (NS * B)) * (NS * B)
    e_pad = ((e + NW * B - 1) // (NW * B)) * (NW * B)
    nch = e_pad // (NW * B)

    src = edge_index[0]
    dst = edge_index[1]
    pad = e_pad - e
    srcp = jnp.concatenate([src, jnp.zeros((pad,), jnp.int32)]).reshape(NW, nch, B)
    dstp = jnp.concatenate([dst, jnp.full((pad,), n, jnp.int32)]).reshape(NW, nch, B)

    x0p = jnp.zeros((n_pad, d), jnp.float32).at[:n].set(x0)
    x1p = jnp.zeros((n_pad, d), jnp.float32).at[:n].set(x1)
    b1 = jnp.concatenate([b1a, b1b]).reshape(1, 2 * h)
    b2r = b2.reshape(1, c_out)

    degp = _deg_sc(nch, n_pad)(dstp)

    zs, dinv = pl.pallas_call(
        _tc1_body,
        out_shape=(
            jax.ShapeDtypeStruct((n_pad, 2 * h), jnp.float32),
            jax.ShapeDtypeStruct((n_pad, 1), jnp.float32),
        ),
    )(x0p, x1p, W1a, W1b, degp)

    p = _scatter_sc(2 * h, nch, n_pad)(srcp, dstp, zs)

    gs = pl.pallas_call(
        _tc2_body,
        out_shape=jax.ShapeDtypeStruct((n_pad, c_out), jnp.float32),
    )(p, zs, dinv, b1, W2)

    q = _scatter_sc(c_out, nch, n_pad)(srcp, dstp, gs)

    out = pl.pallas_call(
        _tc3_body,
        out_shape=jax.ShapeDtypeStruct((n_pad, c_out), jnp.float32),
    )(q, gs, dinv, b2r)

    return out[:n]
